# baseline (speedup 1.0000x reference)
import jax
import jax.numpy as jnp
from jax.experimental import pallas as pl

_K = 1024


def _vq_kernel(x_ref, a_ref, w_ref, wt_ref, c_ref, q_ref, idx_ref, loss_ref):
    xb = x_ref[0]
    w = w_ref[...]
    s = jax.lax.dot_general(
        w, xb, (((1,), (0,)), ((), ())),
        preferred_element_type=jnp.float32)
    a = a_ref[0]
    d = (a - 2.0 * s) + c_ref[...]
    m = jnp.min(d, axis=0, keepdims=True)
    rows = jax.lax.broadcasted_iota(jnp.int32, d.shape, 0)
    idxr = jnp.min(jnp.where(d == m, rows, _K), axis=0, keepdims=True)
    idx_ref[0] = idxr
    ohk = (rows == idxr).astype(jnp.float32)
    q_ref[0] = jnp.dot(wt_ref[...], ohk,
                       preferred_element_type=jnp.float32)

    @pl.when(pl.program_id(0) == 0)
    def _():
        loss_ref[...] = jnp.zeros_like(loss_ref)

    loss_ref[...] += jnp.sum(m, keepdims=True)


def kernel(x, weight, decay, commitment_cost):
    b, c, h, w_ = x.shape
    p = h * w_
    x3 = x.reshape(b, c, p)
    avec = jnp.sum(
        jnp.transpose(x, (0, 2, 3, 1)).reshape(-1, c) ** 2, axis=1
    ).reshape(b, 1, p)
    cvec = jnp.sum(weight**2, axis=1).reshape(_K, 1)
    q, idx, loss = pl.pallas_call(
        _vq_kernel,
        grid=(b,),
        in_specs=[
            pl.BlockSpec((1, c, p), lambda i: (i, 0, 0)),
            pl.BlockSpec((1, 1, p), lambda i: (i, 0, 0)),
            pl.BlockSpec((_K, c), lambda i: (0, 0)),
            pl.BlockSpec((c, _K), lambda i: (0, 0)),
            pl.BlockSpec((_K, 1), lambda i: (0, 0)),
        ],
        out_specs=[
            pl.BlockSpec((1, c, p), lambda i: (i, 0, 0)),
            pl.BlockSpec((1, 1, p), lambda i: (i, 0, 0)),
            pl.BlockSpec((1, 1), lambda i: (0, 0)),
        ],
        out_shape=[
            jax.ShapeDtypeStruct((b, c, p), jnp.float32),
            jax.ShapeDtypeStruct((b, 1, p), jnp.int32),
            jax.ShapeDtypeStruct((1, 1), jnp.float32),
        ],
    )(x3, avec, weight, weight.T, cvec)
    quantized = q.reshape(b, c, h, w_)
    embed_idx = idx.reshape(b, h, w_)
    latent_loss = commitment_cost * (loss[0, 0] / x.size)
    return (quantized, latent_loss, embed_idx)


# speedup vs baseline: 1.4199x; 1.4199x over previous
import jax
import jax.numpy as jnp
from jax.experimental import pallas as pl

_K = 1024


def _vq_kernel(x_ref, w_ref, c_ref, q_ref, idx_ref, loss_ref):
    nb, pp, cc = x_ref.shape
    xp = x_ref[...].reshape(nb * pp, cc)
    w = w_ref[...]
    s = jax.lax.dot_general(
        xp, w, (((1,), (1,)), ((), ())),
        preferred_element_type=jnp.float32)
    a = jnp.sum(xp * xp, axis=1, keepdims=True)
    d = (a - 2.0 * s) + c_ref[...]
    m = jnp.min(d, axis=1, keepdims=True)
    cols = jax.lax.broadcasted_iota(jnp.int32, d.shape, 1)
    idxc = jnp.min(jnp.where(d == m, cols, _K), axis=1, keepdims=True)
    idx_ref[...] = idxc.reshape(nb, pp, 1)
    oh = (cols == idxc).astype(jnp.float32)
    q = jnp.dot(oh, w, preferred_element_type=jnp.float32)
    q_ref[...] = q.reshape(nb, pp, cc)

    @pl.when(pl.program_id(0) == 0)
    def _():
        loss_ref[...] = jnp.zeros_like(loss_ref)

    loss_ref[...] += jnp.sum(m, keepdims=True)


def kernel(x, weight, decay, commitment_cost):
    b, c, h, w_ = x.shape
    p = h * w_
    xt = jnp.transpose(x, (0, 2, 3, 1)).reshape(b, p, c)
    cvec = jnp.sum(weight**2, axis=1).reshape(1, _K)
    q, idx, loss = pl.pallas_call(
        _vq_kernel,
        grid=(b // 2,),
        in_specs=[
            pl.BlockSpec((2, p, c), lambda i: (i, 0, 0)),
            pl.BlockSpec((_K, c), lambda i: (0, 0)),
            pl.BlockSpec((1, _K), lambda i: (0, 0)),
        ],
        out_specs=[
            pl.BlockSpec((2, p, c), lambda i: (i, 0, 0)),
            pl.BlockSpec((2, p, 1), lambda i: (i, 0, 0)),
            pl.BlockSpec((1, 1), lambda i: (0, 0)),
        ],
        out_shape=[
            jax.ShapeDtypeStruct((b, p, c), jnp.float32),
            jax.ShapeDtypeStruct((b, p, 1), jnp.int32),
            jax.ShapeDtypeStruct((1, 1), jnp.float32),
        ],
    )(xt, weight, cvec)
    quantized = jnp.transpose(q.reshape(b, h, w_, c), (0, 3, 1, 2))
    embed_idx = idx.reshape(b, h, w_)
    latent_loss = commitment_cost * (loss[0, 0] / x.size)
    return (quantized, latent_loss, embed_idx)


# speedup vs baseline: 1.4452x; 1.0178x over previous
import jax
import jax.numpy as jnp
from jax.experimental import pallas as pl

_K = 1024


def _vq_kernel(x_ref, w_ref, c_ref, q_ref, idx_ref, loss_ref):
    nb, pp, cc = x_ref.shape
    xp = x_ref[...].reshape(nb * pp, cc)
    w = w_ref[...]
    s = jax.lax.dot_general(
        xp, w, (((1,), (1,)), ((), ())),
        preferred_element_type=jnp.float32)
    a = jnp.sum(xp * xp, axis=1, keepdims=True)
    d = (a - 2.0 * s) + c_ref[...]
    m = jnp.min(d, axis=1, keepdims=True)
    cols = jax.lax.broadcasted_iota(jnp.int32, d.shape, 1)
    idxc = jnp.min(jnp.where(d == m, cols, _K), axis=1, keepdims=True)
    idx_ref[...] = idxc.reshape(nb, pp, 1)
    oh = (cols == idxc).astype(jnp.float32)
    q = jnp.dot(oh, w, preferred_element_type=jnp.float32)
    q_ref[...] = q.reshape(nb, pp, cc)

    @pl.when(pl.program_id(0) == 0)
    def _():
        loss_ref[...] = jnp.zeros_like(loss_ref)

    loss_ref[...] += jnp.sum(m, keepdims=True)


def kernel(x, weight, decay, commitment_cost):
    b, c, h, w_ = x.shape
    p = h * w_
    xt = jnp.transpose(x, (0, 2, 3, 1)).reshape(b, p, c)
    cvec = jnp.sum(weight**2, axis=1).reshape(1, _K)
    q, idx, loss = pl.pallas_call(
        _vq_kernel,
        grid=(b // 4,),
        in_specs=[
            pl.BlockSpec((4, p, c), lambda i: (i, 0, 0)),
            pl.BlockSpec((_K, c), lambda i: (0, 0)),
            pl.BlockSpec((1, _K), lambda i: (0, 0)),
        ],
        out_specs=[
            pl.BlockSpec((4, p, c), lambda i: (i, 0, 0)),
            pl.BlockSpec((4, p, 1), lambda i: (i, 0, 0)),
            pl.BlockSpec((1, 1), lambda i: (0, 0)),
        ],
        out_shape=[
            jax.ShapeDtypeStruct((b, p, c), jnp.float32),
            jax.ShapeDtypeStruct((b, p, 1), jnp.int32),
            jax.ShapeDtypeStruct((1, 1), jnp.float32),
        ],
    )(xt, weight, cvec)
    quantized = jnp.transpose(q.reshape(b, h, w_, c), (0, 3, 1, 2))
    embed_idx = idx.reshape(b, h, w_)
    latent_loss = commitment_cost * (loss[0, 0] / x.size)
    return (quantized, latent_loss, embed_idx)
